# baseline (device time: 95807 ns/iter reference)
import functools

import jax
import jax.numpy as jnp
from jax import lax
from jax.experimental import pallas as pl
from jax.experimental.pallas import tpu as pltpu

N_DEV = 4
E_PER = 4
N_EXP = 16
N_TOK = 1024
D_MODEL = 512
D_HID = 1024


def kernel(x, router_W, route_idx, expert_W):
    def body(x_ref, rw_ref, idx_ref, ew_ref, out_ref,
             comm_ref, send_sems, recv_sems):
        my = lax.axis_index("i")
        left = lax.rem(my + N_DEV - 1, N_DEV)
        right = lax.rem(my + 1, N_DEV)

        barrier_sem = pltpu.get_barrier_semaphore()
        for nbr in (left, right):
            pl.semaphore_signal(
                barrier_sem, inc=1,
                device_id=(nbr,), device_id_type=pl.DeviceIdType.MESH,
            )
        pl.semaphore_wait(barrier_sem, 2)

        xv = x_ref[...]
        scores = jnp.dot(xv, rw_ref[...],
                         preferred_element_type=jnp.float32)
        s_max = jnp.max(scores, axis=-1, keepdims=True)
        p = jnp.exp(scores - s_max)
        probs = p / jnp.sum(p, axis=-1, keepdims=True)

        idx0 = idx_ref[:, 0:1]
        idx1 = idx_ref[:, 1:2]
        iota = lax.broadcasted_iota(jnp.int32, (N_TOK, N_EXP), 1)
        g0 = jnp.sum(jnp.where(iota == idx0, probs, 0.0),
                     axis=-1, keepdims=True)
        g1 = jnp.sum(jnp.where(iota == idx1, probs, 0.0),
                     axis=-1, keepdims=True)
        gs = g0 + g1
        g0 = g0 / gs
        g1 = g1 / gs

        acc = jnp.zeros((N_TOK, D_HID), dtype=jnp.float32)
        for k in range(E_PER):
            e = my * E_PER + k
            w = (jnp.where(idx0 == e, g0, 0.0)
                 + jnp.where(idx1 == e, g1, 0.0))
            xw = (xv * w).astype(jnp.bfloat16)
            acc = acc + jnp.dot(xw, ew_ref[k].astype(jnp.bfloat16),
                                preferred_element_type=jnp.float32)

        out_ref[...] = acc
        comm_ref[0] = acc.astype(jnp.bfloat16)

        for h in range(N_DEV - 1):
            rdma = pltpu.make_async_remote_copy(
                src_ref=comm_ref.at[h],
                dst_ref=comm_ref.at[h + 1],
                send_sem=send_sems.at[h],
                recv_sem=recv_sems.at[h],
                device_id=(right,),
                device_id_type=pl.DeviceIdType.MESH,
            )
            rdma.start()
            rdma.wait()
            out_ref[...] += comm_ref[h + 1][...].astype(jnp.float32)

        @functools.partial(pl.run_scoped,
                           exit_sem=pltpu.SemaphoreType.REGULAR)
        def _(exit_sem):
            for nbr in (left, right):
                pl.semaphore_signal(
                    exit_sem, inc=1,
                    device_id=(nbr,), device_id_type=pl.DeviceIdType.MESH,
                )
            pl.semaphore_wait(exit_sem, 2)

    return pl.pallas_call(
        body,
        out_shape=jax.ShapeDtypeStruct((N_TOK, D_HID), jnp.float32),
        in_specs=[
            pl.BlockSpec(memory_space=pltpu.VMEM),
            pl.BlockSpec(memory_space=pltpu.VMEM),
            pl.BlockSpec(memory_space=pltpu.VMEM),
            pl.BlockSpec(memory_space=pltpu.VMEM),
        ],
        out_specs=pl.BlockSpec(memory_space=pltpu.VMEM),
        scratch_shapes=[
            pltpu.VMEM((N_DEV, N_TOK, D_HID), jnp.bfloat16),
            pltpu.SemaphoreType.DMA((N_DEV - 1,)),
            pltpu.SemaphoreType.DMA((N_DEV - 1,)),
        ],
        compiler_params=pltpu.CompilerParams(collective_id=0),
    )(x, router_W, route_idx, expert_W)


# device time: 46171 ns/iter; 2.0750x vs baseline; 2.0750x over previous
import functools

import jax
import jax.numpy as jnp
from jax import lax
from jax.experimental import pallas as pl
from jax.experimental.pallas import tpu as pltpu

N_DEV = 4
E_PER = 4
N_EXP = 16
N_TOK = 1024
D_MODEL = 512
D_HID = 1024

F32 = jnp.float32
BF16 = jnp.bfloat16


def kernel(x, router_W, route_idx, expert_W):
    def body(x_ref, rw_ref, idx_ref, ew_ref, out_ref,
             s1, r1, s2, r2, s3, r3, s4, r4, send_sems, recv_sems):
        p = lax.axis_index("i")
        pA = p + 1 - 2 * lax.rem(p, 2)
        pB = 3 - p

        barrier_sem = pltpu.get_barrier_semaphore()
        for nbr in (pA, pB):
            pl.semaphore_signal(
                barrier_sem, inc=1,
                device_id=(nbr,), device_id_type=pl.DeviceIdType.MESH,
            )
        pl.semaphore_wait(barrier_sem, 2)

        xv = x_ref[...]
        scores = jnp.dot(xv, rw_ref[...], preferred_element_type=F32)
        s_max = jnp.max(scores, axis=-1, keepdims=True)
        pr = jnp.exp(scores - s_max)
        probs = pr / jnp.sum(pr, axis=-1, keepdims=True)

        idx0 = idx_ref[:, 0:1]
        idx1 = idx_ref[:, 1:2]
        iota = lax.broadcasted_iota(jnp.int32, (N_TOK, N_EXP), 1)
        g0 = jnp.sum(jnp.where(iota == idx0, probs, 0.0),
                     axis=-1, keepdims=True)
        g1 = jnp.sum(jnp.where(iota == idx1, probs, 0.0),
                     axis=-1, keepdims=True)
        gs = g0 + g1
        g0 = g0 / gs
        g1 = g1 / gs

        acc = jnp.zeros((N_TOK, D_HID), dtype=F32)
        for k in range(E_PER):
            e = p * E_PER + k
            w = (jnp.where(idx0 == e, g0, 0.0)
                 + jnp.where(idx1 == e, g1, 0.0))
            xw = (xv * w).astype(BF16)
            acc = acc + jnp.dot(xw, ew_ref[k].astype(BF16),
                                preferred_element_type=F32)
        out_ref[...] = acc

        b01 = jnp.where((p == 1) | (p == 2), 1, 0)
        keep0 = 256 * b01
        send0 = 256 * (1 - b01)
        own0 = jnp.where(p == 0, 0,
               jnp.where(p == 1, 256,
               jnp.where(p == 2, 384, 128)))
        own0_pB = jnp.where(p == 0, 128,
                  jnp.where(p == 1, 384,
                  jnp.where(p == 2, 256, 0)))
        b23 = jnp.where(p >= 2, 1, 0)
        keep1 = 512 + 256 * b23
        send1 = 512 + 256 * (1 - b23)
        own1 = 512 + 128 * p
        own1_pA = 512 + 128 * pA

        def exchange(slot, sbuf, rbuf, nrows, send_off, tgt):
            sbuf[slot] = out_ref[pl.ds(send_off, nrows), :].astype(BF16)
            rdma = pltpu.make_async_remote_copy(
                src_ref=sbuf.at[slot],
                dst_ref=rbuf.at[slot],
                send_sem=send_sems.at[exchange.sem],
                recv_sem=recv_sems.at[exchange.sem],
                device_id=(tgt,),
                device_id_type=pl.DeviceIdType.MESH,
            )
            exchange.sem += 1
            rdma.start()
            return rdma

        exchange.sem = 0

        def add_rows(rbuf, slot, off, nrows):
            out_ref[pl.ds(off, nrows), :] = (
                out_ref[pl.ds(off, nrows), :]
                + rbuf[slot][...].astype(F32))

        def set_rows(rbuf, slot, off, nrows):
            out_ref[pl.ds(off, nrows), :] = rbuf[slot][...].astype(F32)

        d0 = exchange(0, s1, r1, 256, send0, pA)
        d1 = exchange(1, s1, r1, 256, send1, pB)
        d0.wait()
        d1.wait()
        add_rows(r1, 0, keep0, 256)
        add_rows(r1, 1, keep1, 256)

        d0 = exchange(0, s2, r2, 128, own0_pB, pB)
        d1 = exchange(1, s2, r2, 128, own1_pA, pA)
        d0.wait()
        d1.wait()
        add_rows(r2, 0, own0, 128)
        add_rows(r2, 1, own1, 128)

        d0 = exchange(0, s3, r3, 128, own0, pB)
        d1 = exchange(1, s3, r3, 128, own1, pA)
        d0.wait()
        d1.wait()
        set_rows(r3, 0, own0_pB, 128)
        set_rows(r3, 1, own1_pA, 128)

        d0 = exchange(0, s4, r4, 256, keep0, pA)
        d1 = exchange(1, s4, r4, 256, keep1, pB)
        d0.wait()
        d1.wait()
        set_rows(r4, 0, send0, 256)
        set_rows(r4, 1, send1, 256)

        @functools.partial(pl.run_scoped,
                           exit_sem=pltpu.SemaphoreType.REGULAR)
        def _(exit_sem):
            for nbr in (pA, pB):
                pl.semaphore_signal(
                    exit_sem, inc=1,
                    device_id=(nbr,), device_id_type=pl.DeviceIdType.MESH,
                )
            pl.semaphore_wait(exit_sem, 2)

    return pl.pallas_call(
        body,
        out_shape=jax.ShapeDtypeStruct((N_TOK, D_HID), F32),
        in_specs=[
            pl.BlockSpec(memory_space=pltpu.VMEM),
            pl.BlockSpec(memory_space=pltpu.VMEM),
            pl.BlockSpec(memory_space=pltpu.VMEM),
            pl.BlockSpec(memory_space=pltpu.VMEM),
        ],
        out_specs=pl.BlockSpec(memory_space=pltpu.VMEM),
        scratch_shapes=[
            pltpu.VMEM((2, 256, D_HID), BF16),
            pltpu.VMEM((2, 256, D_HID), BF16),
            pltpu.VMEM((2, 128, D_HID), BF16),
            pltpu.VMEM((2, 128, D_HID), BF16),
            pltpu.VMEM((2, 128, D_HID), BF16),
            pltpu.VMEM((2, 128, D_HID), BF16),
            pltpu.VMEM((2, 256, D_HID), BF16),
            pltpu.VMEM((2, 256, D_HID), BF16),
            pltpu.SemaphoreType.DMA((8,)),
            pltpu.SemaphoreType.DMA((8,)),
        ],
        compiler_params=pltpu.CompilerParams(collective_id=0),
    )(x, router_W, route_idx, expert_W)


# device time: 42035 ns/iter; 2.2792x vs baseline; 1.0984x over previous
import jax
import jax.numpy as jnp
from jax import lax
from jax.experimental import pallas as pl
from jax.experimental.pallas import tpu as pltpu

N_DEV = 4
E_PER = 4
N_EXP = 16
N_TOK = 1024
D_MODEL = 512
D_HID = 1024

F32 = jnp.float32
BF16 = jnp.bfloat16


def kernel(x, router_W, route_idx, expert_W):
    def body(x_ref, rw_ref, idx_ref, ew_ref, out_ref,
             ew_bf, g_ref, s1, r1, s2, r2, s3, r3, s4, r4,
             send_sems, recv_sems):
        p = lax.axis_index("i")
        pA = p + 1 - 2 * lax.rem(p, 2)
        pB = 3 - p

        def sel(t):
            return jnp.where(p == 0, t[0],
                   jnp.where(p == 1, t[1],
                   jnp.where(p == 2, t[2], t[3])))

        keep0, send0 = sel([0, 256, 256, 0]), sel([256, 0, 0, 256])
        keep1, send1 = sel([512, 512, 768, 768]), sel([768, 768, 512, 512])
        own0 = sel([0, 256, 384, 128])
        own0_pB = sel([128, 384, 256, 0])
        own1 = 512 + 128 * p
        own1_pA = 512 + 128 * pA
        o0a, o0b = own0_pB - keep0, own0 - keep0
        o1a, o1b = own1_pA - keep1, own1 - keep1

        barrier_sem = pltpu.get_barrier_semaphore()
        for nbr in (pA, pB):
            pl.semaphore_signal(
                barrier_sem, inc=1,
                device_id=(nbr,), device_id_type=pl.DeviceIdType.MESH,
            )
        pl.semaphore_wait(barrier_sem, 2)

        xv = x_ref[...]
        scores = jnp.dot(xv, rw_ref[...], preferred_element_type=F32)
        s_max = jnp.max(scores, axis=-1, keepdims=True)
        pr = jnp.exp(scores - s_max)
        probs = pr / jnp.sum(pr, axis=-1, keepdims=True)
        idx0 = idx_ref[:, 0:1]
        idx1 = idx_ref[:, 1:2]
        iota = lax.broadcasted_iota(jnp.int32, (N_TOK, N_EXP), 1)
        g0 = jnp.sum(jnp.where(iota == idx0, probs, 0.0),
                     axis=-1, keepdims=True)
        g1 = jnp.sum(jnp.where(iota == idx1, probs, 0.0),
                     axis=-1, keepdims=True)
        gs = g0 + g1
        g_ref[:, 0:1] = g0 / gs
        g_ref[:, 1:2] = g1 / gs

        ew_bf[...] = ew_ref[...].astype(BF16)

        def partial_rows(off, n):
            xs = x_ref[pl.ds(off, n), :]
            i0 = idx_ref[pl.ds(off, n), 0:1]
            i1 = idx_ref[pl.ds(off, n), 1:2]
            g0s = g_ref[pl.ds(off, n), 0:1]
            g1s = g_ref[pl.ds(off, n), 1:2]
            acc = jnp.zeros((n, D_HID), dtype=F32)
            for k in range(E_PER):
                e = p * E_PER + k
                w = (jnp.where(i0 == e, g0s, 0.0)
                     + jnp.where(i1 == e, g1s, 0.0))
                xw = (xs * w).astype(BF16)
                acc = acc + jnp.dot(xw, ew_bf[k],
                                    preferred_element_type=F32)
            out_ref[pl.ds(off, n), :] = acc
            return acc

        def start(sem, sbuf, rbuf, slot, tgt):
            rdma = pltpu.make_async_remote_copy(
                src_ref=sbuf.at[slot],
                dst_ref=rbuf.at[slot],
                send_sem=send_sems.at[sem],
                recv_sem=recv_sems.at[sem],
                device_id=(tgt,),
                device_id_type=pl.DeviceIdType.MESH,
            )
            rdma.start()
            return rdma

        acc = partial_rows(send0, 256)
        s1[0] = acc.astype(BF16)
        d1a = start(0, s1, r1, 0, pA)
        acc = partial_rows(send1, 256)
        s1[1] = acc.astype(BF16)
        d1b = start(1, s1, r1, 1, pB)

        partial_rows(own0_pB, 128)
        partial_rows(own1_pA, 128)

        d1a.wait()
        out_ref[pl.ds(own0_pB, 128), :] = (
            out_ref[pl.ds(own0_pB, 128), :]
            + r1[0, pl.ds(o0a, 128), :].astype(F32))
        d1b.wait()
        out_ref[pl.ds(own1_pA, 128), :] = (
            out_ref[pl.ds(own1_pA, 128), :]
            + r1[1, pl.ds(o1a, 128), :].astype(F32))

        s2[0] = out_ref[pl.ds(own0_pB, 128), :].astype(BF16)
        d2a = start(2, s2, r2, 0, pB)
        s2[1] = out_ref[pl.ds(own1_pA, 128), :].astype(BF16)
        d2b = start(3, s2, r2, 1, pA)

        partial_rows(own0, 128)
        partial_rows(own1, 128)
        out_ref[pl.ds(own0, 128), :] = (
            out_ref[pl.ds(own0, 128), :]
            + r1[0, pl.ds(o0b, 128), :].astype(F32))
        out_ref[pl.ds(own1, 128), :] = (
            out_ref[pl.ds(own1, 128), :]
            + r1[1, pl.ds(o1b, 128), :].astype(F32))

        d2a.wait()
        out_ref[pl.ds(own0, 128), :] = (
            out_ref[pl.ds(own0, 128), :] + r2[0][...].astype(F32))
        d2b.wait()
        out_ref[pl.ds(own1, 128), :] = (
            out_ref[pl.ds(own1, 128), :] + r2[1][...].astype(F32))

        s3[0] = out_ref[pl.ds(own0, 128), :].astype(BF16)
        d3a = start(4, s3, r3, 0, pB)
        s3[1] = out_ref[pl.ds(own1, 128), :].astype(BF16)
        d3b = start(5, s3, r3, 1, pA)
        d3a.wait()
        out_ref[pl.ds(own0_pB, 128), :] = r3[0][...].astype(F32)
        d3b.wait()
        out_ref[pl.ds(own1_pA, 128), :] = r3[1][...].astype(F32)

        s4[0] = out_ref[pl.ds(keep0, 256), :].astype(BF16)
        d4a = start(6, s4, r4, 0, pA)
        s4[1] = out_ref[pl.ds(keep1, 256), :].astype(BF16)
        d4b = start(7, s4, r4, 1, pB)
        d4a.wait()
        out_ref[pl.ds(send0, 256), :] = r4[0][...].astype(F32)
        d4b.wait()
        out_ref[pl.ds(send1, 256), :] = r4[1][...].astype(F32)

    return pl.pallas_call(
        body,
        out_shape=jax.ShapeDtypeStruct((N_TOK, D_HID), F32),
        in_specs=[
            pl.BlockSpec(memory_space=pltpu.VMEM),
            pl.BlockSpec(memory_space=pltpu.VMEM),
            pl.BlockSpec(memory_space=pltpu.VMEM),
            pl.BlockSpec(memory_space=pltpu.VMEM),
        ],
        out_specs=pl.BlockSpec(memory_space=pltpu.VMEM),
        scratch_shapes=[
            pltpu.VMEM((E_PER, D_MODEL, D_HID), BF16),
            pltpu.VMEM((N_TOK, 2), F32),
            pltpu.VMEM((2, 256, D_HID), BF16),
            pltpu.VMEM((2, 256, D_HID), BF16),
            pltpu.VMEM((2, 128, D_HID), BF16),
            pltpu.VMEM((2, 128, D_HID), BF16),
            pltpu.VMEM((2, 128, D_HID), BF16),
            pltpu.VMEM((2, 128, D_HID), BF16),
            pltpu.VMEM((2, 256, D_HID), BF16),
            pltpu.VMEM((2, 256, D_HID), BF16),
            pltpu.SemaphoreType.DMA((8,)),
            pltpu.SemaphoreType.DMA((8,)),
        ],
        compiler_params=pltpu.CompilerParams(collective_id=0),
    )(x, router_W, route_idx, expert_W)


# device time: 37976 ns/iter; 2.5228x vs baseline; 1.1069x over previous
import jax
import jax.numpy as jnp
from jax import lax
from jax.experimental import pallas as pl
from jax.experimental.pallas import tpu as pltpu

N_DEV = 4
E_PER = 4
N_EXP = 16
N_TOK = 1024
D_MODEL = 512
D_HID = 1024

F32 = jnp.float32
BF16 = jnp.bfloat16


def kernel(x, router_W, route_idx, expert_W):
    def body(x_ref, rw_ref, idx_ref, ew_hbm, out_ref,
             ew_f32, ew_bf, g_ref, r1, r2,
             copy_sems, send_sems, recv_sems):
        p = lax.axis_index("i")
        pA = p + 1 - 2 * lax.rem(p, 2)
        pB = 3 - p

        def sel(t):
            return jnp.where(p == 0, t[0],
                   jnp.where(p == 1, t[1],
                   jnp.where(p == 2, t[2], t[3])))

        own0 = sel([0, 256, 384, 128])
        own0_pB = sel([128, 384, 256, 0])
        own1 = 512 + 128 * p
        own1_pA = 512 + 128 * pA
        fA = sel([384, 128, 0, 256])
        sA = sel([256, 0, 128, 384])
        fB = sel([768, 896, 512, 640])
        sB = sel([896, 768, 640, 512])

        ew_copies = []
        for k in range(E_PER):
            c = pltpu.make_async_copy(
                ew_hbm.at[k], ew_f32.at[k], copy_sems.at[k])
            c.start()
            ew_copies.append(c)

        barrier_sem = pltpu.get_barrier_semaphore()
        for nbr in (pA, pB):
            pl.semaphore_signal(
                barrier_sem, inc=1,
                device_id=(nbr,), device_id_type=pl.DeviceIdType.MESH,
            )
        pl.semaphore_wait(barrier_sem, 2)

        xv = x_ref[...]
        scores = jnp.dot(xv, rw_ref[...], preferred_element_type=F32)
        s_max = jnp.max(scores, axis=-1, keepdims=True)
        pr = jnp.exp(scores - s_max)
        probs = pr / jnp.sum(pr, axis=-1, keepdims=True)
        idx0 = idx_ref[:, 0:1]
        idx1 = idx_ref[:, 1:2]
        iota = lax.broadcasted_iota(jnp.int32, (N_TOK, N_EXP), 1)
        g0 = jnp.sum(jnp.where(iota == idx0, probs, 0.0),
                     axis=-1, keepdims=True)
        g1 = jnp.sum(jnp.where(iota == idx1, probs, 0.0),
                     axis=-1, keepdims=True)
        gs = g0 + g1
        g_ref[:, 0:1] = g0 / gs
        g_ref[:, 1:2] = g1 / gs

        def partial_rows(off):
            n = 128
            xs = x_ref[pl.ds(off, n), :]
            i0 = idx_ref[pl.ds(off, n), 0:1]
            i1 = idx_ref[pl.ds(off, n), 1:2]
            g0s = g_ref[pl.ds(off, n), 0:1]
            g1s = g_ref[pl.ds(off, n), 1:2]
            acc = jnp.zeros((n, D_HID), dtype=F32)
            for k in range(E_PER):
                e = p * E_PER + k
                w = (jnp.where(i0 == e, g0s, 0.0)
                     + jnp.where(i1 == e, g1s, 0.0))
                xw = (xs * w).astype(BF16)
                acc = acc + jnp.dot(xw, ew_bf[k],
                                    preferred_element_type=F32)
            out_ref[pl.ds(off, n), :] = acc.astype(BF16)

        def rcopy(sem, src, dst, tgt):
            rdma = pltpu.make_async_remote_copy(
                src_ref=src, dst_ref=dst,
                send_sem=send_sems.at[sem],
                recv_sem=recv_sems.at[sem],
                device_id=(tgt,),
                device_id_type=pl.DeviceIdType.MESH,
            )
            rdma.start()
            return rdma

        def add_rows(off, val):
            out_ref[pl.ds(off, 128), :] = (
                out_ref[pl.ds(off, 128), :].astype(F32)
                + val.astype(F32)
            ).astype(BF16)

        for k in range(E_PER):
            ew_copies[k].wait()
            ew_bf[k] = ew_f32[k].astype(BF16)

        partial_rows(fA)
        d1a1 = rcopy(0, out_ref.at[pl.ds(fA, 128)], r1.at[0, 0], pA)
        partial_rows(sA)
        d1a2 = rcopy(1, out_ref.at[pl.ds(sA, 128)], r1.at[0, 1], pA)
        partial_rows(fB)
        d1b1 = rcopy(2, out_ref.at[pl.ds(fB, 128)], r1.at[1, 0], pB)
        partial_rows(sB)
        d1b2 = rcopy(3, out_ref.at[pl.ds(sB, 128)], r1.at[1, 1], pB)

        partial_rows(own0_pB)
        partial_rows(own1_pA)

        d1a1.wait()
        add_rows(own0_pB, r1[0, 0])
        d2a = rcopy(4, out_ref.at[pl.ds(own0_pB, 128)], r2.at[0], pB)
        d1b1.wait()
        add_rows(own1_pA, r1[1, 0])
        d2b = rcopy(5, out_ref.at[pl.ds(own1_pA, 128)], r2.at[1], pA)

        partial_rows(own0)
        partial_rows(own1)
        d1a2.wait()
        add_rows(own0, r1[0, 1])
        d1b2.wait()
        add_rows(own1, r1[1, 1])

        d2a.wait()
        add_rows(own0, r2[0])
        d2b.wait()
        add_rows(own1, r2[1])

        d3a = rcopy(6, out_ref.at[pl.ds(own0, 128)],
                    out_ref.at[pl.ds(own0, 128)], pB)
        d3b = rcopy(7, out_ref.at[pl.ds(own1, 128)],
                    out_ref.at[pl.ds(own1, 128)], pA)

        d4a1 = rcopy(8, out_ref.at[pl.ds(own0, 128)],
                     out_ref.at[pl.ds(own0, 128)], pA)
        d4b1 = rcopy(9, out_ref.at[pl.ds(own1, 128)],
                     out_ref.at[pl.ds(own1, 128)], pB)
        d3a.wait()
        d4a2 = rcopy(10, out_ref.at[pl.ds(own0_pB, 128)],
                     out_ref.at[pl.ds(own0_pB, 128)], pA)
        d3b.wait()
        d4b2 = rcopy(11, out_ref.at[pl.ds(own1_pA, 128)],
                     out_ref.at[pl.ds(own1_pA, 128)], pB)

        d4a1.wait()
        d4a2.wait()
        d4b1.wait()
        d4b2.wait()

    return pl.pallas_call(
        body,
        out_shape=jax.ShapeDtypeStruct((N_TOK, D_HID), BF16),
        in_specs=[
            pl.BlockSpec(memory_space=pltpu.VMEM),
            pl.BlockSpec(memory_space=pltpu.VMEM),
            pl.BlockSpec(memory_space=pltpu.VMEM),
            pl.BlockSpec(memory_space=pl.ANY),
        ],
        out_specs=pl.BlockSpec(memory_space=pltpu.VMEM),
        scratch_shapes=[
            pltpu.VMEM((E_PER, D_MODEL, D_HID), F32),
            pltpu.VMEM((E_PER, D_MODEL, D_HID), BF16),
            pltpu.VMEM((N_TOK, 2), F32),
            pltpu.VMEM((2, 2, 128, D_HID), BF16),
            pltpu.VMEM((2, 128, D_HID), BF16),
            pltpu.SemaphoreType.DMA((E_PER,)),
            pltpu.SemaphoreType.DMA((12,)),
            pltpu.SemaphoreType.DMA((12,)),
        ],
        compiler_params=pltpu.CompilerParams(collective_id=0),
    )(x, router_W, route_idx, expert_W)
